# baseline (speedup 1.0000x reference)
import jax
import jax.numpy as jnp
from jax import lax
from jax.experimental import pallas as pl
from jax.experimental.pallas import tpu as pltpu
from jax.experimental.pallas import tpu_sc as plsc

MAX_DEG = 6
N = 10000
E = 320000
DIN = 128
DHID = 64
NCLS = 2
NB = MAX_DEG + 1

NC = 2
NS = 16
NW = NC * NS
EPT = E // NW
NPAD = 10240
RPT = NPAD // NS


def _seg_sum(feat, src4, dst4, zrows, zdeg, d_feat, with_deg, nbuf, ch, nseg):
    mesh = plsc.VectorSubcoreMesh(core_axis_name="c", subcore_axis_name="s")
    CH, NSEG = ch, nseg
    CPS = EPT // CH // NSEG
    ngrp = CPS // nbuf

    def body(feat_hbm, src_hbm, dst_hbm, zrows_hbm, zdeg_hbm, *rest):
        rest = list(rest)
        hout_hbm = rest.pop(0)
        dout_hbm = rest.pop(0) if with_deg else None
        srcv = rest.pop(0)
        dstv = rest.pop(0)
        rowsbufs = [rest.pop(0) for _ in range(nbuf)]
        onesv = rest.pop(0)
        acc = rest.pop(0)
        dacc = rest.pop(0)
        gsems = [rest.pop(0) for _ in range(nbuf)]
        ssems = [rest.pop(0) for _ in range(nbuf)]
        dsem = rest.pop(0)
        cid = lax.axis_index("c")
        sid = lax.axis_index("s")
        wid = sid * NC + cid

        pltpu.sync_copy(zrows_hbm.at[pl.ds(sid * RPT, RPT)],
                        acc.at[pl.ds(sid * RPT, RPT)])

        @pl.when(sid == 0)
        def _():
            pltpu.sync_copy(zdeg_hbm, dacc)

        for t in range(CH // 16):
            onesv[pl.ds(t * 16, 16)] = jnp.ones((16,), jnp.float32)
        plsc.subcore_barrier()

        def g_start(b, c):
            pltpu.async_copy(feat_hbm.at[srcv.at[c]], rowsbufs[b], gsems[b])

        def g_wait(b, c):
            pltpu.make_async_copy(feat_hbm.at[srcv.at[c]], rowsbufs[b],
                                  gsems[b]).wait()

        def s_start(b, c):
            pltpu.async_copy(rowsbufs[b], acc.at[dstv.at[c]], ssems[b],
                             add=True)

        def s_wait(b, c):
            pltpu.make_async_copy(rowsbufs[b], acc.at[dstv.at[c]],
                                  ssems[b]).wait()

        def d_start(c):
            pltpu.async_copy(onesv, dacc.at[dstv.at[c]], dsem, add=True)

        def handle(b, c):
            g_wait(b, c)
            s_start(b, c)
            if with_deg:
                d_start(c)
            s_wait(b, c)

        def seg_body(seg, carry):
            pltpu.sync_copy(src_hbm.at[wid, seg], srcv)
            pltpu.sync_copy(dst_hbm.at[wid, seg], dstv)
            for b in range(nbuf):
                g_start(b, b)

            def step(j, c2):
                for b in range(nbuf):
                    c = j * nbuf + b
                    handle(b, c)

                    @pl.when(c + nbuf < CPS)
                    def _():
                        g_start(b, c + nbuf)
                return c2

            lax.fori_loop(0, ngrp, step, 0)
            for c in range(ngrp * nbuf, CPS):
                handle(c - ngrp * nbuf, c)
            if with_deg:
                def ddrain(c, c2):
                    pltpu.make_async_copy(onesv, dacc.at[dstv.at[c]],
                                          dsem).wait()
                    return c2

                lax.fori_loop(0, CPS, ddrain, 0)
            return carry

        lax.fori_loop(0, NSEG, seg_body, 0)
        plsc.subcore_barrier()

        pltpu.sync_copy(acc.at[pl.ds(sid * RPT, RPT)],
                        hout_hbm.at[cid, pl.ds(sid * RPT, RPT)])
        if with_deg:
            @pl.when(sid == 0)
            def _():
                pltpu.sync_copy(dacc, dout_hbm.at[cid])

    out_type = [jax.ShapeDtypeStruct((NC, NPAD, d_feat), jnp.float32)]
    if with_deg:
        out_type.append(jax.ShapeDtypeStruct((NC, NPAD), jnp.float32))
    scratch = [
        pltpu.VMEM((CPS, CH), jnp.int32),
        pltpu.VMEM((CPS, CH), jnp.int32),
    ]
    scratch += [pltpu.VMEM((CH, d_feat), jnp.float32) for _ in range(nbuf)]
    scratch += [
        pltpu.VMEM((CH,), jnp.float32),
        pltpu.VMEM_SHARED((NPAD, d_feat), jnp.float32),
        pltpu.VMEM_SHARED((NPAD,), jnp.float32),
    ]
    scratch += [pltpu.SemaphoreType.DMA for _ in range(2 * nbuf + 1)]
    k = pl.kernel(
        body,
        out_type=out_type,
        mesh=mesh,
        scratch_types=scratch,
        compiler_params=pltpu.CompilerParams(use_tc_tiling_on_sc=False),
    )
    return k(feat, src4, dst4, zrows, zdeg)


BLK = 1024


def _dense0_bn(hp, x, degp3, W0, b0, gamma, beta):
    def body(hp_ref, x_ref, deg_ref, w_ref, b_ref, g_ref, be_ref, out_ref,
             y0s, st_ref):
        p = pl.program_id(0)
        i = pl.program_id(1)

        @pl.when(p == 0)
        def _():
            h = hp_ref[0] + hp_ref[1]
            J = jnp.concatenate([h, x_ref[...]], axis=1).astype(jnp.bfloat16)
            P = jnp.dot(J, w_ref[...],
                        preferred_element_type=jnp.float32) + b_ref[...]
            deg = deg_ref[...]
            o = P[:, 0:DHID]
            for d in range(1, NB):
                o = jnp.where(deg >= d, P[:, d * DHID:(d + 1) * DHID], o)
            y0s[pl.ds(i * BLK, BLK), :] = o
            valid = (i * BLK + lax.broadcasted_iota(jnp.int32, (BLK, 1), 0)) < N
            om = jnp.where(valid, o, 0.0)
            st = jnp.concatenate(
                [jnp.sum(om, axis=0, keepdims=True),
                 jnp.sum(om * om, axis=0, keepdims=True)], axis=0)

            @pl.when(i == 0)
            def _():
                st_ref[...] = st

            @pl.when(i > 0)
            def _():
                st_ref[...] = st_ref[...] + st

        @pl.when(p == 1)
        def _():
            mean = st_ref[0:1, :] * (1.0 / N)
            var = st_ref[1:2, :] * (1.0 / N) - mean * mean
            inv = lax.rsqrt(var + 1e-5)
            y = y0s[pl.ds(i * BLK, BLK), :]
            t = (y - mean) * (inv * g_ref[...]) + be_ref[...]
            out_ref[...] = jnp.where(t >= 0, t, 0.01 * t)

    return pl.pallas_call(
        body,
        grid=(2, NPAD // BLK),
        in_specs=[
            pl.BlockSpec((2, BLK, DIN), lambda p, i: (0, i * (1 - p), 0)),
            pl.BlockSpec((BLK, DIN), lambda p, i: (i * (1 - p), 0)),
            pl.BlockSpec((BLK, 1), lambda p, i: (i * (1 - p), 0)),
            pl.BlockSpec((2 * DIN, NB * DHID), lambda p, i: (0, 0)),
            pl.BlockSpec((1, NB * DHID), lambda p, i: (0, 0)),
            pl.BlockSpec((1, DHID), lambda p, i: (0, 0)),
            pl.BlockSpec((1, DHID), lambda p, i: (0, 0)),
        ],
        out_specs=pl.BlockSpec((BLK, DHID), lambda p, i: (i, 0)),
        out_shape=jax.ShapeDtypeStruct((N, DHID), jnp.float32),
        scratch_shapes=[
            pltpu.VMEM((NPAD, DHID), jnp.float32),
            pltpu.VMEM((2, DHID), jnp.float32),
        ],
    )(hp, x, degp3, W0, b0, gamma, beta)


def _dense1(hp, act, degc, W1, b1, finW, finb):
    def body(hp_ref, a_ref, deg_ref, w_ref, b_ref, fw_ref, fb_ref, out_ref):
        h = hp_ref[0] + hp_ref[1]
        J = jnp.concatenate([h, a_ref[...]], axis=1).astype(jnp.bfloat16)
        P = jnp.dot(J, w_ref[...],
                    preferred_element_type=jnp.float32) + b_ref[...]
        deg = deg_ref[...]
        o = P[:, 0:DHID]
        for d in range(1, NB):
            o = jnp.where(deg >= d, P[:, d * DHID:(d + 1) * DHID], o)
        out_ref[...] = jnp.dot(o, fw_ref[...],
                               preferred_element_type=jnp.float32) + fb_ref[...]

    return pl.pallas_call(
        body,
        grid=(NPAD // BLK,),
        in_specs=[
            pl.BlockSpec((2, BLK, DHID), lambda i: (0, i, 0)),
            pl.BlockSpec((BLK, DHID), lambda i: (i, 0)),
            pl.BlockSpec((BLK, 1), lambda i: (i, 0)),
            pl.BlockSpec((2 * DHID, NB * DHID), lambda i: (0, 0)),
            pl.BlockSpec((1, NB * DHID), lambda i: (0, 0)),
            pl.BlockSpec((DHID, NCLS), lambda i: (0, 0)),
            pl.BlockSpec((1, NCLS), lambda i: (0, 0)),
        ],
        out_specs=pl.BlockSpec((BLK, NCLS), lambda i: (i, 0)),
        out_shape=jax.ShapeDtypeStruct((N, NCLS), jnp.float32),
    )(hp, act, degc, W1, b1, finW, finb)


def kernel(x, edge_index, Wl0, bl0, Wr0, Wl1, bl1, Wr1,
           bn0_gamma, bn0_beta, fin_W, fin_b):
    ei = edge_index.astype(jnp.int32)
    src0 = ei[0].reshape(NW, 10, 25, 40)
    dst0 = ei[1].reshape(NW, 10, 25, 40)
    src1 = ei[0].reshape(NW, 5, 25, 80)
    dst1 = ei[1].reshape(NW, 5, 25, 80)
    zrows0 = jnp.zeros((NPAD, DIN), jnp.float32)
    zrows1 = jnp.zeros((NPAD, DHID), jnp.float32)
    zdeg = jnp.zeros((NPAD,), jnp.float32)

    hp0, degp = _seg_sum(x, src0, dst0, zrows0, zdeg, DIN, True, 5, 40, 10)
    degc = (degp[0] + degp[1]).reshape(NPAD, 1)

    W0 = jnp.concatenate(
        [Wl0.transpose(1, 0, 2).reshape(DIN, NB * DHID),
         Wr0.transpose(1, 0, 2).reshape(DIN, NB * DHID)],
        axis=0).astype(jnp.bfloat16)
    b0 = bl0.reshape(1, NB * DHID)
    act = _dense0_bn(hp0, x, degc, W0, b0,
                     bn0_gamma.reshape(1, DHID), bn0_beta.reshape(1, DHID))

    hp1, = _seg_sum(act, src1, dst1, zrows1, zdeg, DHID, False, 6, 80, 5)

    W1 = jnp.concatenate(
        [Wl1.transpose(1, 0, 2).reshape(DHID, NB * DHID),
         Wr1.transpose(1, 0, 2).reshape(DHID, NB * DHID)],
        axis=0).astype(jnp.bfloat16)
    b1 = bl1.reshape(1, NB * DHID)
    out = _dense1(hp1, act, degc, W1, b1, fin_W, fin_b.reshape(1, NCLS))
    return out


# speedup vs baseline: 1.0529x; 1.0529x over previous
import jax
import jax.numpy as jnp
from jax import lax
from jax.experimental import pallas as pl
from jax.experimental.pallas import tpu as pltpu
from jax.experimental.pallas import tpu_sc as plsc

MAX_DEG = 6
N = 10000
E = 320000
DIN = 128
DHID = 64
NCLS = 2
NB = MAX_DEG + 1

NC = 2
NS = 16
NW = NC * NS
EPT = E // NW
NPAD = 10240
RPT = NPAD // NS


def _seg_sum(feat, src4, dst4, zrows, zdeg, d_feat, with_deg, nbuf, ch, nseg):
    mesh = plsc.VectorSubcoreMesh(core_axis_name="c", subcore_axis_name="s")
    CH, NSEG = ch, nseg
    CPS = EPT // CH // NSEG
    ngrp = CPS // nbuf

    def body(feat_hbm, src_hbm, dst_hbm, zrows_hbm, zdeg_hbm, *rest):
        rest = list(rest)
        hout_hbm = rest.pop(0)
        dout_hbm = rest.pop(0) if with_deg else None
        srcv = rest.pop(0)
        dstv = rest.pop(0)
        rowsbufs = [rest.pop(0) for _ in range(nbuf)]
        onesv = rest.pop(0)
        acc = rest.pop(0)
        dacc = rest.pop(0)
        gsems = [rest.pop(0) for _ in range(nbuf)]
        ssems = [rest.pop(0) for _ in range(nbuf)]
        dsem = rest.pop(0)
        cid = lax.axis_index("c")
        sid = lax.axis_index("s")
        wid = sid * NC + cid

        pltpu.sync_copy(zrows_hbm.at[pl.ds(sid * RPT, RPT)],
                        acc.at[pl.ds(sid * RPT, RPT)])

        @pl.when(sid == 0)
        def _():
            pltpu.sync_copy(zdeg_hbm, dacc)

        for t in range(CH // 16):
            onesv[pl.ds(t * 16, 16)] = jnp.ones((16,), jnp.float32)
        plsc.subcore_barrier()

        def g_start(b, c):
            pltpu.async_copy(feat_hbm.at[srcv.at[c]], rowsbufs[b], gsems[b])

        def g_wait(b, c):
            pltpu.make_async_copy(feat_hbm.at[srcv.at[c]], rowsbufs[b],
                                  gsems[b]).wait()

        def s_start(b, c):
            pltpu.async_copy(rowsbufs[b], acc.at[dstv.at[c]], ssems[b],
                             add=True)

        def s_wait(b, c):
            pltpu.make_async_copy(rowsbufs[b], acc.at[dstv.at[c]],
                                  ssems[b]).wait()

        def d_start(c):
            pltpu.async_copy(onesv, dacc.at[dstv.at[c]], dsem, add=True)

        def handle(b, c):
            g_wait(b, c)
            s_start(b, c)
            if with_deg:
                d_start(c)
            s_wait(b, c)

        def seg_body(seg, carry):
            pltpu.sync_copy(src_hbm.at[wid, seg], srcv)
            pltpu.sync_copy(dst_hbm.at[wid, seg], dstv)
            for b in range(nbuf):
                g_start(b, b)

            def step(j, c2):
                for b in range(nbuf):
                    c = j * nbuf + b
                    handle(b, c)

                    @pl.when(c + nbuf < CPS)
                    def _():
                        g_start(b, c + nbuf)
                return c2

            lax.fori_loop(0, ngrp, step, 0)
            for c in range(ngrp * nbuf, CPS):
                handle(c - ngrp * nbuf, c)
            if with_deg:
                def ddrain(c, c2):
                    pltpu.make_async_copy(onesv, dacc.at[dstv.at[c]],
                                          dsem).wait()
                    return c2

                lax.fori_loop(0, CPS, ddrain, 0)
            return carry

        lax.fori_loop(0, NSEG, seg_body, 0)
        plsc.subcore_barrier()

        pltpu.sync_copy(acc.at[pl.ds(sid * RPT, RPT)],
                        hout_hbm.at[cid, pl.ds(sid * RPT, RPT)])
        if with_deg:
            @pl.when(sid == 0)
            def _():
                pltpu.sync_copy(dacc, dout_hbm.at[cid])

    out_type = [jax.ShapeDtypeStruct((NC, NPAD, d_feat), jnp.float32)]
    if with_deg:
        out_type.append(jax.ShapeDtypeStruct((NC, NPAD), jnp.float32))
    scratch = [
        pltpu.VMEM((CPS, CH), jnp.int32),
        pltpu.VMEM((CPS, CH), jnp.int32),
    ]
    scratch += [pltpu.VMEM((CH, d_feat), jnp.float32) for _ in range(nbuf)]
    scratch += [
        pltpu.VMEM((CH,), jnp.float32),
        pltpu.VMEM_SHARED((NPAD, d_feat), jnp.float32),
        pltpu.VMEM_SHARED((NPAD,), jnp.float32),
    ]
    scratch += [pltpu.SemaphoreType.DMA for _ in range(2 * nbuf + 1)]
    k = pl.kernel(
        body,
        out_type=out_type,
        mesh=mesh,
        scratch_types=scratch,
        compiler_params=pltpu.CompilerParams(use_tc_tiling_on_sc=False),
    )
    return k(feat, src4, dst4, zrows, zdeg)


BLK = 2000


def _dense0_bn(hp, x, degp3, W0, b0, gamma, beta):
    def body(hp_ref, x_ref, deg_ref, w_ref, b_ref, g_ref, be_ref, out_ref,
             y0s, st_ref):
        p = pl.program_id(0)
        i = pl.program_id(1)

        @pl.when(p == 0)
        def _():
            h = hp_ref[0] + hp_ref[1]
            J = jnp.concatenate([h, x_ref[...]], axis=1).astype(jnp.bfloat16)
            P = jnp.dot(J, w_ref[...],
                        preferred_element_type=jnp.float32) + b_ref[...]
            deg = deg_ref[...]
            o = P[:, 0:DHID]
            for d in range(1, NB):
                o = jnp.where(deg >= d, P[:, d * DHID:(d + 1) * DHID], o)
            y0s[pl.ds(i * BLK, BLK), :] = o
            st = jnp.concatenate(
                [jnp.sum(o, axis=0, keepdims=True),
                 jnp.sum(o * o, axis=0, keepdims=True)], axis=0)

            @pl.when(i == 0)
            def _():
                st_ref[...] = st

            @pl.when(i > 0)
            def _():
                st_ref[...] = st_ref[...] + st

        @pl.when(p == 1)
        def _():
            mean = st_ref[0:1, :] * (1.0 / N)
            var = st_ref[1:2, :] * (1.0 / N) - mean * mean
            inv = lax.rsqrt(var + 1e-5)
            y = y0s[pl.ds(i * BLK, BLK), :]
            t = (y - mean) * (inv * g_ref[...]) + be_ref[...]
            out_ref[...] = jnp.where(t >= 0, t, 0.01 * t)

    return pl.pallas_call(
        body,
        grid=(2, N // BLK),
        in_specs=[
            pl.BlockSpec((2, BLK, DIN), lambda p, i: (0, i * (1 - p), 0)),
            pl.BlockSpec((BLK, DIN), lambda p, i: (i * (1 - p), 0)),
            pl.BlockSpec((BLK, 1), lambda p, i: (i * (1 - p), 0)),
            pl.BlockSpec((2 * DIN, NB * DHID), lambda p, i: (0, 0)),
            pl.BlockSpec((1, NB * DHID), lambda p, i: (0, 0)),
            pl.BlockSpec((1, DHID), lambda p, i: (0, 0)),
            pl.BlockSpec((1, DHID), lambda p, i: (0, 0)),
        ],
        out_specs=pl.BlockSpec((BLK, DHID), lambda p, i: (i, 0)),
        out_shape=jax.ShapeDtypeStruct((N, DHID), jnp.float32),
        scratch_shapes=[
            pltpu.VMEM((N, DHID), jnp.float32),
            pltpu.VMEM((2, DHID), jnp.float32),
        ],
    )(hp, x, degp3, W0, b0, gamma, beta)


def _dense1(hp, act, degc, W1, b1, finW, finb):
    def body(hp_ref, a_ref, deg_ref, w_ref, b_ref, fw_ref, fb_ref, out_ref):
        h = hp_ref[0] + hp_ref[1]
        J = jnp.concatenate([h, a_ref[...]], axis=1).astype(jnp.bfloat16)
        P = jnp.dot(J, w_ref[...],
                    preferred_element_type=jnp.float32) + b_ref[...]
        deg = deg_ref[...]
        o = P[:, 0:DHID]
        for d in range(1, NB):
            o = jnp.where(deg >= d, P[:, d * DHID:(d + 1) * DHID], o)
        out_ref[...] = jnp.dot(o, fw_ref[...],
                               preferred_element_type=jnp.float32) + fb_ref[...]

    return pl.pallas_call(
        body,
        grid=(N // BLK,),
        in_specs=[
            pl.BlockSpec((2, BLK, DHID), lambda i: (0, i, 0)),
            pl.BlockSpec((BLK, DHID), lambda i: (i, 0)),
            pl.BlockSpec((BLK, 1), lambda i: (i, 0)),
            pl.BlockSpec((2 * DHID, NB * DHID), lambda i: (0, 0)),
            pl.BlockSpec((1, NB * DHID), lambda i: (0, 0)),
            pl.BlockSpec((DHID, NCLS), lambda i: (0, 0)),
            pl.BlockSpec((1, NCLS), lambda i: (0, 0)),
        ],
        out_specs=pl.BlockSpec((BLK, NCLS), lambda i: (i, 0)),
        out_shape=jax.ShapeDtypeStruct((N, NCLS), jnp.float32),
    )(hp, act, degc, W1, b1, finW, finb)


def kernel(x, edge_index, Wl0, bl0, Wr0, Wl1, bl1, Wr1,
           bn0_gamma, bn0_beta, fin_W, fin_b):
    ei = edge_index.astype(jnp.int32)
    src1 = ei[0].reshape(NW, 5, 25, 80)
    dst1 = ei[1].reshape(NW, 5, 25, 80)
    zrows0 = jnp.zeros((NPAD, DIN), jnp.float32)
    zrows1 = jnp.zeros((NPAD, DHID), jnp.float32)
    zdeg = jnp.zeros((NPAD,), jnp.float32)

    hp0, degp = _seg_sum(x, src1, dst1, zrows0, zdeg, DIN, True, 3, 80, 5)
    degc = (degp[0] + degp[1]).reshape(NPAD, 1)

    W0 = jnp.concatenate(
        [Wl0.transpose(1, 0, 2).reshape(DIN, NB * DHID),
         Wr0.transpose(1, 0, 2).reshape(DIN, NB * DHID)],
        axis=0).astype(jnp.bfloat16)
    b0 = bl0.reshape(1, NB * DHID)
    act = _dense0_bn(hp0, x, degc, W0, b0,
                     bn0_gamma.reshape(1, DHID), bn0_beta.reshape(1, DHID))

    hp1, = _seg_sum(act, src1, dst1, zrows1, zdeg, DHID, False, 6, 80, 5)

    W1 = jnp.concatenate(
        [Wl1.transpose(1, 0, 2).reshape(DHID, NB * DHID),
         Wr1.transpose(1, 0, 2).reshape(DHID, NB * DHID)],
        axis=0).astype(jnp.bfloat16)
    b1 = bl1.reshape(1, NB * DHID)
    out = _dense1(hp1, act, degc, W1, b1, fin_W, fin_b.reshape(1, NCLS))
    return out
